# baseline (speedup 1.0000x reference)
import jax
import jax.numpy as jnp
from jax import lax
from jax.experimental import pallas as pl
from jax.experimental.pallas import tpu as pltpu
from jax.experimental.pallas import tpu_sc as plsc

OBJ = 1024
ATTR = 8
DIM = 256
BATCH = 4
SEQ = OBJ * ATTR

_info = plsc.get_sparse_core_info()
_NC, _NS, _L = _info.num_cores, _info.num_subcores, _info.num_lanes
_NW = _NC * _NS
_ROWS_W = SEQ // _NW
_TROWS_W = _ROWS_W // ATTR

_CH = 256


def _sc_body(table_hbm, out_hbm, tbuf, rows_v, wsem):
    wid = lax.axis_index("s") * _NC + lax.axis_index("c")
    base = wid * _ROWS_W
    pltpu.sync_copy(table_hbm.at[pl.ds(wid * _TROWS_W, _TROWS_W)], tbuf)

    def exp_row(r, carry):
        for k in range(DIM // _L):
            v = tbuf[r, pl.ds(k * _L, _L)]
            for a in range(ATTR):
                rows_v[r * ATTR + a, pl.ds(k * _L, _L)] = v
        return carry

    lax.fori_loop(0, _TROWS_W, exp_row, 0)
    pltpu.async_copy(rows_v, out_hbm.at[0, pl.ds(base, _ROWS_W)], wsem).wait()


def _tc_body(table_ref, out_ref, eb, sem):
    cps = []
    for j in range(OBJ // _CH):
        t = table_ref[pl.ds(j * _CH, _CH), :]
        lo = j * _CH * ATTR
        eb[pl.ds(lo, _CH * ATTR), :] = jnp.broadcast_to(
            t[:, None, :], (_CH, ATTR, DIM)
        ).reshape(_CH * ATTR, DIM)
        for b in range(BATCH):
            c = pltpu.make_async_copy(
                eb.at[pl.ds(lo, _CH * ATTR)],
                out_ref.at[b, pl.ds(lo, _CH * ATTR)],
                sem,
            )
            c.start()
            cps.append(c)
    for c in cps:
        c.wait()


def kernel(x, E_object_index):
    del x
    tc_out = pl.pallas_call(
        _tc_body,
        in_specs=[pl.BlockSpec((OBJ, DIM), lambda: (0, 0))],
        out_specs=pl.BlockSpec(memory_space=pl.ANY),
        out_shape=jax.ShapeDtypeStruct((BATCH, SEQ, DIM), jnp.float32),
        scratch_shapes=[
            pltpu.VMEM((SEQ, DIM), jnp.float32),
            pltpu.SemaphoreType.DMA,
        ],
    )(E_object_index)
    return tc_out


# speedup vs baseline: 1.0378x; 1.0378x over previous
import jax
import jax.numpy as jnp
from jax import lax
from jax.experimental import pallas as pl
from jax.experimental.pallas import tpu as pltpu
from jax.experimental.pallas import tpu_sc as plsc

OBJ = 1024
ATTR = 8
DIM = 256
BATCH = 4
SEQ = OBJ * ATTR

_info = plsc.get_sparse_core_info()
_NC, _NS, _L = _info.num_cores, _info.num_subcores, _info.num_lanes
_NW = _NC * _NS
_ROWS_W = SEQ // _NW
_TROWS_W = _ROWS_W // ATTR

_CH = 64


def _sc_body(table_hbm, out_hbm, tbuf, rows_v, wsem):
    wid = lax.axis_index("s") * _NC + lax.axis_index("c")
    base = wid * _ROWS_W
    pltpu.sync_copy(table_hbm.at[pl.ds(wid * _TROWS_W, _TROWS_W)], tbuf)

    def exp_row(r, carry):
        for k in range(DIM // _L):
            v = tbuf[r, pl.ds(k * _L, _L)]
            for a in range(ATTR):
                rows_v[r * ATTR + a, pl.ds(k * _L, _L)] = v
        return carry

    lax.fori_loop(0, _TROWS_W, exp_row, 0)
    pltpu.async_copy(rows_v, out_hbm.at[0, pl.ds(base, _ROWS_W)], wsem).wait()


def _tc_body(table_ref, out_ref, eb, sem):
    cps = []
    for j in range(OBJ // _CH):
        t = table_ref[pl.ds(j * _CH, _CH), :]
        lo = j * _CH * ATTR
        eb[pl.ds(lo, _CH * ATTR), :] = jnp.broadcast_to(
            t[:, None, :], (_CH, ATTR, DIM)
        ).reshape(_CH * ATTR, DIM)
        for b in range(BATCH):
            c = pltpu.make_async_copy(
                eb.at[pl.ds(lo, _CH * ATTR)],
                out_ref.at[b, pl.ds(lo, _CH * ATTR)],
                sem,
            )
            c.start()
            cps.append(c)
    for c in cps:
        c.wait()


def kernel(x, E_object_index):
    del x
    tc_out = pl.pallas_call(
        _tc_body,
        in_specs=[pl.BlockSpec((OBJ, DIM), lambda: (0, 0))],
        out_specs=pl.BlockSpec(memory_space=pl.ANY),
        out_shape=jax.ShapeDtypeStruct((BATCH, SEQ, DIM), jnp.float32),
        scratch_shapes=[
            pltpu.VMEM((SEQ, DIM), jnp.float32),
            pltpu.SemaphoreType.DMA,
        ],
    )(E_object_index)
    return tc_out
